# baseline (speedup 1.0000x reference)
import functools

import jax
import jax.numpy as jnp
from jax import lax
from jax.experimental import pallas as pl
from jax.experimental.pallas import tpu as pltpu
from jax.experimental.pallas import tpu_sc as plsc

N = 10000
E = 160000
D_IN = 16
D_H = 16
D_EDGE = 4

NC = 2
NS = 16
NW = NC * NS
PER_TILE = E // NW
C = 500
NCHUNK = PER_TILE // C
NPAD = ((N + NS - 1) // NS + 7) // 8 * 8 * NS
ZR = NPAD // NS
CR = N // 16


def _make_sc_edge_pass(with_count: bool):
    mesh = plsc.VectorSubcoreMesh(core_axis_name="c", subcore_axis_name="s")

    out_type = [jax.ShapeDtypeStruct((NC, NPAD, 16), jnp.float32)]
    scratch = [
        pltpu.VMEM((2, C), jnp.int32),
        pltpu.VMEM((2, C), jnp.int32),
        pltpu.VMEM((2, C * D_EDGE), jnp.float32),
        pltpu.VMEM((2, C, 80), jnp.float32),
        pltpu.VMEM((2, C, 16), jnp.float32),
        pltpu.VMEM_SHARED((NPAD, 16), jnp.float32),
        pltpu.SemaphoreType.DMA((2,)),
        pltpu.SemaphoreType.DMA((2,)),
    ]
    if with_count:
        out_type.append(jax.ShapeDtypeStruct((NC, NS, CR, 16), jnp.float32))
        scratch.append(pltpu.VMEM((CR, 16), jnp.float32))

    @functools.partial(
        pl.kernel,
        out_type=out_type,
        mesh=mesh,
        compiler_params=pltpu.CompilerParams(use_tc_tiling_on_sc=False,
                                             needs_layout_passes=False),
        scratch_types=scratch,
    )
    def sc_pass(u_hbm, src_hbm, dst_hbm, ea_hbm, z_hbm, *rest):
        if with_count:
            (out_hbm, cnt_out, src_v, dst_v, ea_v, rows_v, msg_v, acc_sh,
             sem_i, sem_g, cnt_v) = rest
        else:
            (out_hbm, src_v, dst_v, ea_v, rows_v, msg_v, acc_sh,
             sem_i, sem_g) = rest
        cid = lax.axis_index("c")
        sid = lax.axis_index("s")
        base0 = (cid * NS + sid) * NCHUNK

        def issue_idx(j, b):
            row = base0 + j
            return (
                pltpu.async_copy(src_hbm.at[row], src_v.at[b], sem_i.at[b]),
                pltpu.async_copy(dst_hbm.at[row], dst_v.at[b], sem_i.at[b]),
                pltpu.async_copy(ea_hbm.at[row], ea_v.at[b], sem_i.at[b]),
            )

        idx_d0 = issue_idx(0, 0)
        idx_d1 = issue_idx(1, 1)
        pltpu.sync_copy(z_hbm, acc_sh.at[pl.ds(sid * ZR, ZR)])
        if with_count:
            zv = jnp.zeros((16,), jnp.float32)

            @plsc.parallel_loop(0, CR, step=1, unroll=8)
            def zstep(i):
                cnt_v[i, pl.ds(0, 16)] = zv

            ones16 = jnp.ones((16,), jnp.float32)
        plsc.subcore_barrier()
        for d in idx_d0:
            d.wait()
        pltpu.async_copy(u_hbm.at[src_v.at[0]], rows_v.at[0], sem_g.at[0])

        def wait_idx(b):
            row = base0
            pltpu.make_async_copy(src_hbm.at[row], src_v.at[b],
                                  sem_i.at[b]).wait()
            pltpu.make_async_copy(dst_hbm.at[row], dst_v.at[b],
                                  sem_i.at[b]).wait()
            pltpu.make_async_copy(ea_hbm.at[row], ea_v.at[b],
                                  sem_i.at[b]).wait()

        def body(j2, carry):
            for b in (0, 1):
                nb = 1 - b
                j = j2 * 2 + b
                pltpu.make_async_copy(u_hbm.at[src_v.at[b]], rows_v.at[b],
                                      sem_g.at[b]).wait()

                @pl.when(j + 1 < NCHUNK)
                def _():
                    wait_idx(nb)
                    pltpu.async_copy(u_hbm.at[src_v.at[nb]], rows_v.at[nb],
                                     sem_g.at[nb])

                if with_count:
                    @plsc.parallel_loop(0, C // 16, step=1, unroll=4)
                    def hstep(g):
                        d = dst_v[b, pl.ds(g * 16, 16)]
                        row = lax.shift_right_logical(d, 4)
                        col = jnp.bitwise_and(d, 15)
                        plsc.addupdate_scatter(cnt_v, [row, col], ones16)

                    if C % 16:
                        d = dst_v[b, pl.ds(C - 16, 16)]
                        row = lax.shift_right_logical(d, 4)
                        col = jnp.bitwise_and(d, 15)
                        plsc.addupdate_scatter(
                            cnt_v, [row, col], ones16,
                            mask=lax.iota(jnp.int32, 16) >= (16 - C % 16))

                @plsc.parallel_loop(0, C // 4, step=1, unroll=4)
                def step(g):
                    eav = ea_v[b, pl.ds(g * 16, 16)]
                    for i in range(4):
                        e = g * 4 + i
                        m = rows_v[b, e, pl.ds(64, 16)]
                        for k in range(4):
                            a = eav.at[jnp.full((16,), 4 * i + k,
                                                 jnp.int32)].get(
                                mode="promise_in_bounds")
                            m = m + a * rows_v[b, e, pl.ds(k * 16, 16)]
                        msg_v[b, e, pl.ds(0, 16)] = m

                pltpu.sync_copy(msg_v.at[b], acc_sh.at[dst_v.at[b]], add=True)

                @pl.when(j + 2 < NCHUNK)
                def _():
                    issue_idx(j + 2, b)
            return carry

        lax.fori_loop(0, NCHUNK // 2, body, 0)

        plsc.subcore_barrier()
        pltpu.sync_copy(acc_sh.at[pl.ds(sid * ZR, ZR)],
                        out_hbm.at[cid, pl.ds(sid * ZR, ZR)])
        if with_count:
            pltpu.sync_copy(cnt_v, cnt_out.at[cid, sid])

    return sc_pass


_sc_pass_l0 = _make_sc_edge_pass(with_count=True)
_sc_pass_l1 = _make_sc_edge_pass(with_count=False)

_DOT = dict(precision=lax.Precision.HIGHEST, preferred_element_type=jnp.float32)
_TC_PARAMS = pltpu.CompilerParams(vmem_limit_bytes=100 * 2**20)


def _tc_in(x_ref, wc_ref, root_ref, bias_ref, u_ref, r_ref):
    x = x_ref[...]
    u_ref[...] = jnp.dot(x, wc_ref[...], **_DOT)
    r_ref[...] = jnp.dot(x, root_ref[...], **_DOT) + bias_ref[...]


def _mean(s, inv2):
    return (s.reshape(CR, 16, 16) * inv2[:, :, None]).reshape(N, 16)


def _tc_mid(sp_ref, cnt_ref, r0_ref, wc_ref, root_ref, bias_ref,
            u_ref, r_ref, inv_ref):
    c = jnp.sum(cnt_ref[...], axis=(0, 1))
    inv2 = 1.0 / jnp.maximum(c, 1.0)
    s = (sp_ref[0] + sp_ref[1])[:N]
    h = jnp.maximum(r0_ref[...] + _mean(s, inv2), 0.0)
    u_ref[...] = jnp.dot(h, wc_ref[...], **_DOT)
    r_ref[...] = jnp.dot(h, root_ref[...], **_DOT) + bias_ref[...]
    inv_ref[...] = inv2


def _tc_out(sp_ref, r1_ref, inv_ref, hw_ref, hb_ref, out_ref):
    s = (sp_ref[0] + sp_ref[1])[:N]
    h = jnp.maximum(r1_ref[...] + _mean(s, inv_ref[...]), 0.0)
    out_ref[...] = jnp.dot(h, hw_ref[...], **_DOT) + hb_ref[...]


def _wcat(w, b):
    wk = w.reshape(D_EDGE, 16, 16).transpose(1, 0, 2).reshape(16, D_EDGE * 16)
    return jnp.concatenate([wk, b.reshape(16, 16)], axis=1)


def kernel(x, edge_index, edge_attr, enn0_w, enn0_b, root0, bias0,
           enn1_w, enn1_b, root1, bias1, head_w, head_b):
    src = edge_index[0].astype(jnp.int32).reshape(E // C, C)
    dst = edge_index[1].astype(jnp.int32).reshape(E // C, C)
    ea_flat = edge_attr.reshape(E // C, C * D_EDGE)
    wc0 = _wcat(enn0_w, enn0_b)
    wc1 = _wcat(enn1_w, enn1_b)
    z16 = jnp.zeros((ZR, 16), jnp.float32)

    u0, r0 = pl.pallas_call(
        _tc_in,
        compiler_params=_TC_PARAMS,
        out_shape=[jax.ShapeDtypeStruct((N, 80), jnp.float32),
                   jax.ShapeDtypeStruct((N, 16), jnp.float32)],
    )(x, wc0, root0, bias0.reshape(1, 16))

    s0, c0 = _sc_pass_l0(u0, src, dst, ea_flat, z16)

    u1, r1, inv2 = pl.pallas_call(
        _tc_mid,
        compiler_params=_TC_PARAMS,
        out_shape=[jax.ShapeDtypeStruct((N, 80), jnp.float32),
                   jax.ShapeDtypeStruct((N, 16), jnp.float32),
                   jax.ShapeDtypeStruct((CR, 16), jnp.float32)],
    )(s0, c0, r0, wc1, root1, bias1.reshape(1, 16))

    s1 = _sc_pass_l1(u1, src, dst, ea_flat, z16)
    if isinstance(s1, (list, tuple)):
        s1 = s1[0]

    out = pl.pallas_call(
        _tc_out,
        compiler_params=_TC_PARAMS,
        out_shape=jax.ShapeDtypeStruct((N, 1), jnp.float32),
    )(s1, r1, inv2, head_w, head_b.reshape(1, 1))
    return out


# speedup vs baseline: 1.0021x; 1.0021x over previous
import functools

import jax
import jax.numpy as jnp
from jax import lax
from jax.experimental import pallas as pl
from jax.experimental.pallas import tpu as pltpu
from jax.experimental.pallas import tpu_sc as plsc

N = 10000
E = 160000
D_IN = 16
D_H = 16
D_EDGE = 4

NC = 2
NS = 16
NW = NC * NS
PER_TILE = E // NW
C = 500
NCHUNK = PER_TILE // C
NPAD = ((N + NS - 1) // NS + 7) // 8 * 8 * NS
ZR = NPAD // NS
CR = N // 16


def _make_sc_edge_pass(with_count: bool):
    mesh = plsc.VectorSubcoreMesh(core_axis_name="c", subcore_axis_name="s")

    out_type = [jax.ShapeDtypeStruct((NC, NPAD, 16), jnp.float32)]
    scratch = [
        pltpu.VMEM((2, C), jnp.int32),
        pltpu.VMEM((2, C), jnp.int32),
        pltpu.VMEM((2, C * D_EDGE), jnp.float32),
        pltpu.VMEM((2, C, 80), jnp.float32),
        pltpu.VMEM((2, C, 16), jnp.float32),
        pltpu.VMEM_SHARED((NPAD, 16), jnp.float32),
        pltpu.SemaphoreType.DMA((2,)),
        pltpu.SemaphoreType.DMA((2,)),
    ]
    if with_count:
        out_type.append(jax.ShapeDtypeStruct((NC, NS, CR, 16), jnp.float32))
        scratch.append(pltpu.VMEM((CR, 16), jnp.float32))

    @functools.partial(
        pl.kernel,
        out_type=out_type,
        mesh=mesh,
        compiler_params=pltpu.CompilerParams(use_tc_tiling_on_sc=False,
                                             needs_layout_passes=False),
        scratch_types=scratch,
    )
    def sc_pass(u_hbm, src_hbm, dst_hbm, ea_hbm, z_hbm, *rest):
        if with_count:
            (out_hbm, cnt_out, src_v, dst_v, ea_v, rows_v, msg_v, acc_sh,
             sem_i, sem_g, cnt_v) = rest
        else:
            (out_hbm, src_v, dst_v, ea_v, rows_v, msg_v, acc_sh,
             sem_i, sem_g) = rest
        cid = lax.axis_index("c")
        sid = lax.axis_index("s")
        base0 = (cid * NS + sid) * NCHUNK

        def issue_idx(j, b):
            row = base0 + j
            return (
                pltpu.async_copy(src_hbm.at[row], src_v.at[b], sem_i.at[b]),
                pltpu.async_copy(dst_hbm.at[row], dst_v.at[b], sem_i.at[b]),
                pltpu.async_copy(ea_hbm.at[row], ea_v.at[b], sem_i.at[b]),
            )

        idx_d0 = issue_idx(0, 0)
        idx_d1 = issue_idx(1, 1)
        pltpu.sync_copy(z_hbm, acc_sh.at[pl.ds(sid * ZR, ZR)])
        if with_count:
            zv = jnp.zeros((16,), jnp.float32)

            @plsc.parallel_loop(0, CR, step=1, unroll=8)
            def zstep(i):
                cnt_v[i, pl.ds(0, 16)] = zv

            ones16 = jnp.ones((16,), jnp.float32)
        plsc.subcore_barrier()
        for d in idx_d0:
            d.wait()
        pltpu.async_copy(u_hbm.at[src_v.at[0]], rows_v.at[0], sem_g.at[0])

        def wait_idx(b):
            row = base0
            pltpu.make_async_copy(src_hbm.at[row], src_v.at[b],
                                  sem_i.at[b]).wait()
            pltpu.make_async_copy(dst_hbm.at[row], dst_v.at[b],
                                  sem_i.at[b]).wait()
            pltpu.make_async_copy(ea_hbm.at[row], ea_v.at[b],
                                  sem_i.at[b]).wait()

        def body(j2, carry):
            for b in (0, 1):
                nb = 1 - b
                j = j2 * 2 + b
                pltpu.make_async_copy(u_hbm.at[src_v.at[b]], rows_v.at[b],
                                      sem_g.at[b]).wait()

                @pl.when(j + 1 < NCHUNK)
                def _():
                    wait_idx(nb)
                    pltpu.async_copy(u_hbm.at[src_v.at[nb]], rows_v.at[nb],
                                     sem_g.at[nb])

                if with_count:
                    @plsc.parallel_loop(0, C // 16, step=1, unroll=4)
                    def hstep(g):
                        d = dst_v[b, pl.ds(g * 16, 16)]
                        row = lax.shift_right_logical(d, 4)
                        col = jnp.bitwise_and(d, 15)
                        plsc.addupdate_scatter(cnt_v, [row, col], ones16)

                    if C % 16:
                        d = dst_v[b, pl.ds(C - 16, 16)]
                        row = lax.shift_right_logical(d, 4)
                        col = jnp.bitwise_and(d, 15)
                        plsc.addupdate_scatter(
                            cnt_v, [row, col], ones16,
                            mask=lax.iota(jnp.int32, 16) >= (16 - C % 16))

                @plsc.parallel_loop(0, C // 4, step=1, unroll=4)
                def step(g):
                    eav = ea_v[b, pl.ds(g * 16, 16)]
                    for i in range(4):
                        e = g * 4 + i
                        m = rows_v[b, e, pl.ds(64, 16)]
                        m = m + eav[4 * i + 0] * rows_v[b, e, pl.ds(0, 16)]
                        m = m + eav[4 * i + 1] * rows_v[b, e, pl.ds(16, 16)]
                        m = m + eav[4 * i + 2] * rows_v[b, e, pl.ds(32, 16)]
                        m = m + eav[4 * i + 3] * rows_v[b, e, pl.ds(48, 16)]
                        msg_v[b, e, pl.ds(0, 16)] = m

                pltpu.sync_copy(msg_v.at[b], acc_sh.at[dst_v.at[b]], add=True)

                @pl.when(j + 2 < NCHUNK)
                def _():
                    issue_idx(j + 2, b)
            return carry

        lax.fori_loop(0, NCHUNK // 2, body, 0)

        plsc.subcore_barrier()
        pltpu.sync_copy(acc_sh.at[pl.ds(sid * ZR, ZR)],
                        out_hbm.at[cid, pl.ds(sid * ZR, ZR)])
        if with_count:
            pltpu.sync_copy(cnt_v, cnt_out.at[cid, sid])

    return sc_pass


_sc_pass_l0 = _make_sc_edge_pass(with_count=True)
_sc_pass_l1 = _make_sc_edge_pass(with_count=False)

_DOT = dict(precision=lax.Precision.HIGHEST, preferred_element_type=jnp.float32)
_TC_PARAMS = pltpu.CompilerParams(vmem_limit_bytes=100 * 2**20)


def _tc_in(x_ref, wc_ref, root_ref, bias_ref, u_ref, r_ref):
    x = x_ref[...]
    u_ref[...] = jnp.dot(x, wc_ref[...], **_DOT)
    r_ref[...] = jnp.dot(x, root_ref[...], **_DOT) + bias_ref[...]


def _mean(s, inv2):
    return (s.reshape(CR, 16, 16) * inv2[:, :, None]).reshape(N, 16)


def _tc_mid(sp_ref, cnt_ref, r0_ref, wc_ref, root_ref, bias_ref,
            u_ref, r_ref, inv_ref):
    c = jnp.sum(cnt_ref[...], axis=(0, 1))
    inv2 = 1.0 / jnp.maximum(c, 1.0)
    s = (sp_ref[0] + sp_ref[1])[:N]
    h = jnp.maximum(r0_ref[...] + _mean(s, inv2), 0.0)
    u_ref[...] = jnp.dot(h, wc_ref[...], **_DOT)
    r_ref[...] = jnp.dot(h, root_ref[...], **_DOT) + bias_ref[...]
    inv_ref[...] = inv2


def _tc_out(sp_ref, r1_ref, inv_ref, hw_ref, hb_ref, out_ref):
    s = (sp_ref[0] + sp_ref[1])[:N]
    h = jnp.maximum(r1_ref[...] + _mean(s, inv_ref[...]), 0.0)
    out_ref[...] = jnp.dot(h, hw_ref[...], **_DOT) + hb_ref[...]


def _wcat(w, b):
    wk = w.reshape(D_EDGE, 16, 16).transpose(1, 0, 2).reshape(16, D_EDGE * 16)
    return jnp.concatenate([wk, b.reshape(16, 16)], axis=1)


def kernel(x, edge_index, edge_attr, enn0_w, enn0_b, root0, bias0,
           enn1_w, enn1_b, root1, bias1, head_w, head_b):
    src = edge_index[0].astype(jnp.int32).reshape(E // C, C)
    dst = edge_index[1].astype(jnp.int32).reshape(E // C, C)
    ea_flat = edge_attr.reshape(E // C, C * D_EDGE)
    wc0 = _wcat(enn0_w, enn0_b)
    wc1 = _wcat(enn1_w, enn1_b)
    z16 = jnp.zeros((ZR, 16), jnp.float32)

    u0, r0 = pl.pallas_call(
        _tc_in,
        compiler_params=_TC_PARAMS,
        out_shape=[jax.ShapeDtypeStruct((N, 80), jnp.float32),
                   jax.ShapeDtypeStruct((N, 16), jnp.float32)],
    )(x, wc0, root0, bias0.reshape(1, 16))

    s0, c0 = _sc_pass_l0(u0, src, dst, ea_flat, z16)

    u1, r1, inv2 = pl.pallas_call(
        _tc_mid,
        compiler_params=_TC_PARAMS,
        out_shape=[jax.ShapeDtypeStruct((N, 80), jnp.float32),
                   jax.ShapeDtypeStruct((N, 16), jnp.float32),
                   jax.ShapeDtypeStruct((CR, 16), jnp.float32)],
    )(s0, c0, r0, wc1, root1, bias1.reshape(1, 16))

    s1 = _sc_pass_l1(u1, src, dst, ea_flat, z16)
    if isinstance(s1, (list, tuple)):
        s1 = s1[0]

    out = pl.pallas_call(
        _tc_out,
        compiler_params=_TC_PARAMS,
        out_shape=jax.ShapeDtypeStruct((N, 1), jnp.float32),
    )(s1, r1, inv2, head_w, head_b.reshape(1, 1))
    return out
